# Initial kernel scaffold; baseline (speedup 1.0000x reference)
import jax
import jax.numpy as jnp
from jax.experimental import pallas as pl


def kernel(x, edge_index, edge_type, W1, root1, b1, W2, root2, b2):
    raise NotImplementedError("write your pallas kernel here")


# speedup vs baseline: 5.0224x; 5.0224x over previous
import jax
import jax.numpy as jnp
from jax import lax
from jax.experimental import pallas as pl
from jax.experimental.pallas import tpu as pltpu
from jax.experimental.pallas import tpu_sc as plsc

N_NODES = 10000
N_EDGES = 320000
DIM = 128
N_RELS = 8

N_CHUNKS = 16
CHUNK_NODES = 625
CHUNK_SLOTS = 640
N_SLOTS = N_CHUNKS * CHUNK_SLOTS
LOCAL_ROWS = CHUNK_SLOTS * N_RELS
OUT_ROWS = N_CHUNKS * LOCAL_ROWS
STRIPE = LOCAL_ROWS // 16
PASSES = N_CHUNKS // 2
E_PER_TILE = N_EDGES // 16
E_STAGE = 2000
N_STAGE = E_PER_TILE // E_STAGE
VECS_PER_STAGE = E_STAGE // 16
COMP_ROWS = 32
DUMMY_ROW = CHUNK_NODES * N_RELS + 100
DUMMY_SRC = CHUNK_NODES

_TC_BLK = 1280


def _sc_scatter_body(x_pad, esrc, edst, ety, z_h, a_out,
                     est_s, est_d, est_t, csrc, ckey, rowbuf, zbuf, acc):
    c = lax.axis_index("c")
    s = lax.axis_index("s")
    iota = lax.iota(jnp.int32, 16)

    pltpu.sync_copy(z_h, zbuf)

    def drain(lo, hi):
        def gloop(j, carry):
            row = lax.bitwise_and(j, COMP_ROWS - 1)
            pltpu.sync_copy(x_pad.at[csrc.at[row]], rowbuf)
            pltpu.sync_copy(rowbuf, acc.at[ckey.at[row]], add=True)
            return carry
        lax.fori_loop(lo, hi, gloop, 0)

    def pass_body(p, _):
        chunk = c * PASSES + p
        node_base = chunk * CHUNK_NODES
        out_base = chunk * LOCAL_ROWS

        def zloop(k, carry):
            pltpu.sync_copy(zbuf, acc.at[pl.ds(s * STRIPE + k * 32, 32)])
            return carry
        lax.fori_loop(0, STRIPE // 32, zloop, 0)
        plsc.subcore_barrier()

        def stage(b, carry):
            off, drained = carry
            e0 = s * E_PER_TILE + b * E_STAGE
            pltpu.sync_copy(esrc.at[pl.ds(e0, E_STAGE)], est_s)
            pltpu.sync_copy(edst.at[pl.ds(e0, E_STAGE)], est_d)
            pltpu.sync_copy(ety.at[pl.ds(e0, E_STAGE)], est_t)

            def iloop(i, off):
                sl = pl.ds(i * 16, 16)
                sv = est_s[sl]
                dv = est_d[sl]
                tv = est_t[sl]
                dr = dv - node_base
                m = (dr >= 0) & (dr < CHUNK_NODES)
                key = dr * N_RELS + tv
                pos = plsc.cumsum(jnp.where(m, 1, 0).astype(jnp.int32))
                idx = off + pos - 1
                row = lax.bitwise_and(
                    lax.shift_right_logical(idx, 7), COMP_ROWS - 1)
                col = lax.bitwise_and(idx, 127)
                plsc.store_scatter(csrc, [row, col], sv, mask=m)
                plsc.store_scatter(ckey, [row, col], key, mask=m)
                return off + plsc.all_reduce_population_count(m)
            off = lax.fori_loop(0, VECS_PER_STAGE, iloop, off)
            full = lax.shift_right_logical(jnp.max(off), 7)
            drain(drained, full)
            return (off, full)
        off, drained = lax.fori_loop(
            0, N_STAGE, stage, (jnp.zeros((16,), jnp.int32),
                                jnp.zeros((), jnp.int32)))
        n_edges = jnp.max(off)
        nb = lax.shift_right_logical(n_edges + 127, 7)

        def ploop(j, carry):
            posv = n_edges + j * 16 + iota
            pm = posv < nb * 128
            row = lax.bitwise_and(
                lax.shift_right_logical(posv, 7), COMP_ROWS - 1)
            col = lax.bitwise_and(posv, 127)
            plsc.store_scatter(
                ckey, [row, col],
                jnp.full((16,), DUMMY_ROW, jnp.int32), mask=pm)
            plsc.store_scatter(
                csrc, [row, col],
                jnp.full((16,), DUMMY_SRC, jnp.int32), mask=pm)
            return carry
        lax.fori_loop(0, 8, ploop, 0)
        drain(drained, nb)
        plsc.subcore_barrier()

        def wloop(k, carry):
            r0 = s * STRIPE + k * 64
            rb = rowbuf.at[pl.ds(0, 64)]
            pltpu.sync_copy(acc.at[pl.ds(r0, 64)], rb)
            pltpu.sync_copy(rb, a_out.at[pl.ds(out_base + r0, 64)])
            return carry
        lax.fori_loop(0, STRIPE // 64, wloop, 0)
        return 0

    lax.fori_loop(0, PASSES, pass_body, 0)


def _make_sc_scatter():
    mesh = plsc.VectorSubcoreMesh(core_axis_name="c", subcore_axis_name="s",
                                  num_cores=2, num_subcores=16)
    out_type = [jax.ShapeDtypeStruct((OUT_ROWS, DIM), jnp.float32)]
    scratch = [
        pltpu.VMEM((E_STAGE,), jnp.int32),
        pltpu.VMEM((E_STAGE,), jnp.int32),
        pltpu.VMEM((E_STAGE,), jnp.int32),
        pltpu.VMEM((COMP_ROWS, 128), jnp.int32),
        pltpu.VMEM((COMP_ROWS, 128), jnp.int32),
        pltpu.VMEM((128, DIM), jnp.float32),
        pltpu.VMEM((32, DIM), jnp.float32),
        pltpu.VMEM_SHARED((LOCAL_ROWS, DIM), jnp.float32),
    ]
    return pl.kernel(_sc_scatter_body,
                     out_type=out_type, mesh=mesh, scratch_types=scratch,
                     compiler_params=pltpu.CompilerParams(
                         needs_layout_passes=False))


def _dense_body(x_ref, a_ref, cnt_ref, w_ref, b_ref, o_ref):
    x = x_ref[...]
    acc = jnp.dot(x, w_ref[0:DIM, :], preferred_element_type=jnp.float32)
    acc = acc + b_ref[0, :][None, :]
    for r in range(N_RELS):
        inv = 1.0 / jnp.maximum(cnt_ref[:, r], 1.0)
        ar = a_ref[:, r, :] * inv[:, None]
        acc = acc + jnp.dot(
            ar, w_ref[DIM * (r + 1):DIM * (r + 2), :],
            preferred_element_type=jnp.float32)
    o_ref[...] = jnp.maximum(acc, 0.0)


def _dense_layer(x, a, cnt, wcat, bias, *, interpret=False):
    return pl.pallas_call(
        _dense_body,
        grid=(N_SLOTS // _TC_BLK,),
        in_specs=[
            pl.BlockSpec((_TC_BLK, DIM), lambda i: (i, 0)),
            pl.BlockSpec((_TC_BLK, N_RELS, DIM), lambda i: (i, 0, 0)),
            pl.BlockSpec((_TC_BLK, N_RELS), lambda i: (i, 0)),
            pl.BlockSpec(((N_RELS + 1) * DIM, DIM), lambda i: (0, 0)),
            pl.BlockSpec((1, DIM), lambda i: (0, 0)),
        ],
        out_specs=pl.BlockSpec((_TC_BLK, DIM), lambda i: (i, 0)),
        out_shape=jax.ShapeDtypeStruct((N_SLOTS, DIM), jnp.float32),
        interpret=interpret,
    )(x, a, cnt, wcat, bias)


def kernel(x, edge_index, edge_type, W1, root1, b1, W2, root2, b2):
    src = edge_index[0].astype(jnp.int32)
    dst = edge_index[1].astype(jnp.int32)
    ety = edge_type.astype(jnp.int32)
    src_slot = (src // CHUNK_NODES) * CHUNK_SLOTS + src % CHUNK_NODES

    x_pad = jnp.pad(x.reshape(N_CHUNKS, CHUNK_NODES, DIM),
                    ((0, 0), (0, CHUNK_SLOTS - CHUNK_NODES), (0, 0))
                    ).reshape(N_SLOTS, DIM)

    w1 = jnp.concatenate([root1, W1.reshape(N_RELS * DIM, DIM)], axis=0)
    w2 = jnp.concatenate([root2, W2.reshape(N_RELS * DIM, DIM)], axis=0)
    bias1 = b1.reshape(1, DIM)
    bias2 = b2.reshape(1, DIM)

    z_h = jnp.zeros((32, DIM), jnp.float32)
    ones_mat = jnp.ones((N_SLOTS, DIM), jnp.float32)

    scat = _make_sc_scatter()

    (cnt_wide,) = scat(ones_mat, src_slot, dst, ety, z_h)
    cnt = cnt_wide[:, 0].reshape(N_SLOTS, N_RELS)

    (a1f,) = scat(x_pad, src_slot, dst, ety, z_h)
    a1 = a1f.reshape(N_SLOTS, N_RELS, DIM)
    h = _dense_layer(x_pad, a1, cnt, w1, bias1)

    (a2f,) = scat(h, src_slot, dst, ety, z_h)
    a2 = a2f.reshape(N_SLOTS, N_RELS, DIM)
    h2 = _dense_layer(h, a2, cnt, w2, bias2)

    return h2.reshape(N_CHUNKS, CHUNK_SLOTS, DIM)[:, :CHUNK_NODES].reshape(
        N_NODES, DIM)


# speedup vs baseline: 8.1742x; 1.6276x over previous
import jax
import jax.numpy as jnp
from jax import lax
from jax.experimental import pallas as pl
from jax.experimental.pallas import tpu as pltpu
from jax.experimental.pallas import tpu_sc as plsc

N_NODES = 10000
N_EDGES = 320000
DIM = 128
N_RELS = 8

N_CHUNKS = 10
CHUNK_NODES = 1000
CHUNK_SLOTS = 1024
N_SLOTS = N_CHUNKS * CHUNK_SLOTS
LOCAL_ROWS = CHUNK_SLOTS * N_RELS
OUT_ROWS = N_CHUNKS * LOCAL_ROWS
STRIPE = LOCAL_ROWS // 16
PASSES = N_CHUNKS // 2
E_PER_TILE = N_EDGES // 16
E_STAGE = 2000
N_STAGE = E_PER_TILE // E_STAGE
VECS_PER_STAGE = E_STAGE // 16
COMP_ROWS = 32
DUMMY_ROW = CHUNK_NODES * N_RELS + 100
DUMMY_SRC = CHUNK_NODES

_TC_BLK = 1280


def _sc_scatter_body(gather, x_pad, esrc, edst, ety, z_h, o_h, a_out,
                     est_s, est_d, est_t, csrc, ckey, rowbuf, zbuf, acc):
    c = lax.axis_index("c")
    s = lax.axis_index("s")
    iota = lax.iota(jnp.int32, 16)

    pltpu.sync_copy(z_h, zbuf)

    def drain(lo, hi):
        def gloop(j, carry):
            row = lax.bitwise_and(j, COMP_ROWS - 1)
            if gather:
                pltpu.sync_copy(x_pad.at[csrc.at[row]], rowbuf)
            pltpu.sync_copy(rowbuf, acc.at[ckey.at[row]], add=True)
            return carry
        lax.fori_loop(lo, hi, gloop, 0)

    def pass_body(p, _):
        chunk = c * PASSES + p
        node_base = chunk * CHUNK_NODES
        out_base = chunk * LOCAL_ROWS
        if not gather:
            pltpu.sync_copy(o_h, rowbuf)

        def zloop(k, carry):
            pltpu.sync_copy(zbuf, acc.at[pl.ds(s * STRIPE + k * 32, 32)])
            return carry
        lax.fori_loop(0, STRIPE // 32, zloop, 0)
        plsc.subcore_barrier()

        def stage(b, carry):
            off, drained = carry
            e0 = s * E_PER_TILE + b * E_STAGE
            pltpu.sync_copy(esrc.at[pl.ds(e0, E_STAGE)], est_s)
            pltpu.sync_copy(edst.at[pl.ds(e0, E_STAGE)], est_d)
            pltpu.sync_copy(ety.at[pl.ds(e0, E_STAGE)], est_t)

            def iloop(i, off):
                sl = pl.ds(i * 16, 16)
                sv = est_s[sl]
                dv = est_d[sl]
                tv = est_t[sl]
                dr = dv - node_base
                m = (dr >= 0) & (dr < CHUNK_NODES)
                key = dr * N_RELS + tv
                pos = plsc.cumsum(jnp.where(m, 1, 0).astype(jnp.int32))
                idx = off + pos - 1
                row = lax.bitwise_and(
                    lax.shift_right_logical(idx, 7), COMP_ROWS - 1)
                col = lax.bitwise_and(idx, 127)
                plsc.store_scatter(csrc, [row, col], sv, mask=m)
                plsc.store_scatter(ckey, [row, col], key, mask=m)
                return off + plsc.all_reduce_population_count(m)
            off = lax.fori_loop(0, VECS_PER_STAGE, iloop, off)
            full = lax.shift_right_logical(jnp.max(off), 7)
            drain(drained, full)
            return (off, full)
        off, drained = lax.fori_loop(
            0, N_STAGE, stage, (jnp.zeros((16,), jnp.int32),
                                jnp.zeros((), jnp.int32)))
        n_edges = jnp.max(off)
        nb = lax.shift_right_logical(n_edges + 127, 7)

        def ploop(j, carry):
            posv = n_edges + j * 16 + iota
            pm = posv < nb * 128
            row = lax.bitwise_and(
                lax.shift_right_logical(posv, 7), COMP_ROWS - 1)
            col = lax.bitwise_and(posv, 127)
            plsc.store_scatter(
                ckey, [row, col],
                jnp.full((16,), DUMMY_ROW, jnp.int32), mask=pm)
            plsc.store_scatter(
                csrc, [row, col],
                jnp.full((16,), DUMMY_SRC, jnp.int32), mask=pm)
            return carry
        lax.fori_loop(0, 8, ploop, 0)
        drain(drained, nb)
        plsc.subcore_barrier()

        def wloop(k, carry):
            r0 = s * STRIPE + k * 64
            rb = rowbuf.at[pl.ds(0, 64)]
            pltpu.sync_copy(acc.at[pl.ds(r0, 64)], rb)
            pltpu.sync_copy(rb, a_out.at[pl.ds(out_base + r0, 64)])
            return carry
        lax.fori_loop(0, STRIPE // 64, wloop, 0)
        return 0

    lax.fori_loop(0, PASSES, pass_body, 0)


def _make_sc_scatter(gather=True):
    mesh = plsc.VectorSubcoreMesh(core_axis_name="c", subcore_axis_name="s",
                                  num_cores=2, num_subcores=16)
    out_type = [jax.ShapeDtypeStruct((OUT_ROWS, DIM), jnp.float32)]
    scratch = [
        pltpu.VMEM((E_STAGE,), jnp.int32),
        pltpu.VMEM((E_STAGE,), jnp.int32),
        pltpu.VMEM((E_STAGE,), jnp.int32),
        pltpu.VMEM((COMP_ROWS, 128), jnp.int32),
        pltpu.VMEM((COMP_ROWS, 128), jnp.int32),
        pltpu.VMEM((128, DIM), jnp.float32),
        pltpu.VMEM((32, DIM), jnp.float32),
        pltpu.VMEM_SHARED((LOCAL_ROWS, DIM), jnp.float32),
    ]
    import functools as _ft
    return pl.kernel(_ft.partial(_sc_scatter_body, gather),
                     out_type=out_type, mesh=mesh, scratch_types=scratch,
                     compiler_params=pltpu.CompilerParams(
                         needs_layout_passes=False))


def _dense_body(x_ref, a_ref, cnt_ref, w_ref, b_ref, o_ref):
    x = x_ref[...]
    acc = jnp.dot(x, w_ref[0:DIM, :], preferred_element_type=jnp.float32)
    acc = acc + b_ref[0, :][None, :]
    for r in range(N_RELS):
        inv = 1.0 / jnp.maximum(cnt_ref[:, r], 1.0)
        ar = a_ref[:, r, :] * inv[:, None]
        acc = acc + jnp.dot(
            ar, w_ref[DIM * (r + 1):DIM * (r + 2), :],
            preferred_element_type=jnp.float32)
    o_ref[...] = jnp.maximum(acc, 0.0)


def _dense_layer(x, a, cnt, wcat, bias, *, interpret=False):
    return pl.pallas_call(
        _dense_body,
        grid=(N_SLOTS // _TC_BLK,),
        in_specs=[
            pl.BlockSpec((_TC_BLK, DIM), lambda i: (i, 0)),
            pl.BlockSpec((_TC_BLK, N_RELS, DIM), lambda i: (i, 0, 0)),
            pl.BlockSpec((_TC_BLK, N_RELS), lambda i: (i, 0)),
            pl.BlockSpec(((N_RELS + 1) * DIM, DIM), lambda i: (0, 0)),
            pl.BlockSpec((1, DIM), lambda i: (0, 0)),
        ],
        out_specs=pl.BlockSpec((_TC_BLK, DIM), lambda i: (i, 0)),
        out_shape=jax.ShapeDtypeStruct((N_SLOTS, DIM), jnp.float32),
        interpret=interpret,
    )(x, a, cnt, wcat, bias)


def kernel(x, edge_index, edge_type, W1, root1, b1, W2, root2, b2):
    src = edge_index[0].astype(jnp.int32)
    dst = edge_index[1].astype(jnp.int32)
    ety = edge_type.astype(jnp.int32)
    src_slot = (src // CHUNK_NODES) * CHUNK_SLOTS + src % CHUNK_NODES

    x_pad = jnp.pad(x.reshape(N_CHUNKS, CHUNK_NODES, DIM),
                    ((0, 0), (0, CHUNK_SLOTS - CHUNK_NODES), (0, 0))
                    ).reshape(N_SLOTS, DIM)

    w1 = jnp.concatenate([root1, W1.reshape(N_RELS * DIM, DIM)], axis=0)
    w2 = jnp.concatenate([root2, W2.reshape(N_RELS * DIM, DIM)], axis=0)
    bias1 = b1.reshape(1, DIM)
    bias2 = b2.reshape(1, DIM)

    z_h = jnp.zeros((32, DIM), jnp.float32)
    o_h = jnp.ones((128, DIM), jnp.float32)

    scat = _make_sc_scatter(True)
    scat_cnt = _make_sc_scatter(False)

    (cnt_wide,) = scat_cnt(x_pad, src_slot, dst, ety, z_h, o_h)
    cnt = cnt_wide[:, 0].reshape(N_SLOTS, N_RELS)

    (a1f,) = scat(x_pad, src_slot, dst, ety, z_h, o_h)
    a1 = a1f.reshape(N_SLOTS, N_RELS, DIM)
    h = _dense_layer(x_pad, a1, cnt, w1, bias1)

    (a2f,) = scat(h, src_slot, dst, ety, z_h, o_h)
    a2 = a2f.reshape(N_SLOTS, N_RELS, DIM)
    h2 = _dense_layer(h, a2, cnt, w2, bias2)

    return h2.reshape(N_CHUNKS, CHUNK_SLOTS, DIM)[:, :CHUNK_NODES].reshape(
        N_NODES, DIM)


# speedup vs baseline: 8.6562x; 1.0590x over previous
import jax
import jax.numpy as jnp
from jax import lax
from jax.experimental import pallas as pl
from jax.experimental.pallas import tpu as pltpu
from jax.experimental.pallas import tpu_sc as plsc

N_NODES = 10000
N_EDGES = 320000
DIM = 128
N_RELS = 8

N_CHUNKS = 10
CHUNK_NODES = 1000
CHUNK_SLOTS = 1024
N_SLOTS = N_CHUNKS * CHUNK_SLOTS
LOCAL_ROWS = CHUNK_SLOTS * N_RELS
OUT_ROWS = N_CHUNKS * LOCAL_ROWS
STRIPE = LOCAL_ROWS // 16
PASSES = N_CHUNKS // 2
E_PER_TILE = N_EDGES // 16
E_STAGE = 4000
N_STAGE = E_PER_TILE // E_STAGE
VECS_PER_STAGE = E_STAGE // 16
COMP_ROWS = 64
DUMMY_ROW = CHUNK_NODES * N_RELS + 100
DUMMY_SRC = CHUNK_NODES

_TC_BLK = 1280


def _sc_scatter_body(gather, x_pad, esrc, edst, ety, z_h, o_h, a_out,
                     est_s, est_d, est_t, csrc, ckey, rowbuf, zbuf, acc):
    c = lax.axis_index("c")
    s = lax.axis_index("s")
    iota = lax.iota(jnp.int32, 16)

    pltpu.sync_copy(z_h, zbuf)

    def drain(lo, hi):
        def gloop(j, carry):
            row = lax.bitwise_and(j, COMP_ROWS - 1)
            if gather:
                pltpu.sync_copy(x_pad.at[csrc.at[row]], rowbuf)
            pltpu.sync_copy(rowbuf, acc.at[ckey.at[row]], add=True)
            return carry
        lax.fori_loop(lo, hi, gloop, 0)

    def pass_body(p, _):
        chunk = c * PASSES + p
        node_base = chunk * CHUNK_NODES
        out_base = chunk * LOCAL_ROWS
        if not gather:
            pltpu.sync_copy(o_h, rowbuf)

        def zloop(k, carry):
            pltpu.sync_copy(zbuf, acc.at[pl.ds(s * STRIPE + k * 32, 32)])
            return carry
        lax.fori_loop(0, STRIPE // 32, zloop, 0)
        plsc.subcore_barrier()

        def stage(b, carry):
            off, drained = carry
            e0 = s * E_PER_TILE + b * E_STAGE
            pltpu.sync_copy(esrc.at[pl.ds(e0, E_STAGE)], est_s)
            pltpu.sync_copy(edst.at[pl.ds(e0, E_STAGE)], est_d)
            pltpu.sync_copy(ety.at[pl.ds(e0, E_STAGE)], est_t)

            def iloop(i, off):
                for u in range(5):
                    sl = pl.ds((i * 5 + u) * 16, 16)
                    sv = est_s[sl]
                    dv = est_d[sl]
                    tv = est_t[sl]
                    dr = dv - node_base
                    m = (dr >= 0) & (dr < CHUNK_NODES)
                    key = dr * N_RELS + tv
                    pos = plsc.cumsum(jnp.where(m, 1, 0).astype(jnp.int32))
                    idx = off + pos - 1
                    row = lax.bitwise_and(
                        lax.shift_right_logical(idx, 7), COMP_ROWS - 1)
                    col = lax.bitwise_and(idx, 127)
                    plsc.store_scatter(csrc, [row, col], sv, mask=m)
                    plsc.store_scatter(ckey, [row, col], key, mask=m)
                    off = off + plsc.all_reduce_population_count(m)
                return off
            off = lax.fori_loop(0, VECS_PER_STAGE // 5, iloop, off)
            full = lax.shift_right_logical(jnp.max(off), 7)
            drain(drained, full)
            return (off, full)
        off, drained = lax.fori_loop(
            0, N_STAGE, stage, (jnp.zeros((16,), jnp.int32),
                                jnp.zeros((), jnp.int32)))
        n_edges = jnp.max(off)
        nb = lax.shift_right_logical(n_edges + 127, 7)

        def ploop(j, carry):
            posv = n_edges + j * 16 + iota
            pm = posv < nb * 128
            row = lax.bitwise_and(
                lax.shift_right_logical(posv, 7), COMP_ROWS - 1)
            col = lax.bitwise_and(posv, 127)
            plsc.store_scatter(
                ckey, [row, col],
                jnp.full((16,), DUMMY_ROW, jnp.int32), mask=pm)
            plsc.store_scatter(
                csrc, [row, col],
                jnp.full((16,), DUMMY_SRC, jnp.int32), mask=pm)
            return carry
        lax.fori_loop(0, 8, ploop, 0)
        drain(drained, nb)
        plsc.subcore_barrier()

        def wloop(k, carry):
            r0 = s * STRIPE + k * 64
            rb = rowbuf.at[pl.ds(0, 64)]
            pltpu.sync_copy(acc.at[pl.ds(r0, 64)], rb)
            pltpu.sync_copy(rb, a_out.at[pl.ds(out_base + r0, 64)])
            return carry
        lax.fori_loop(0, STRIPE // 64, wloop, 0)
        return 0

    lax.fori_loop(0, PASSES, pass_body, 0)


def _make_sc_scatter(gather=True):
    mesh = plsc.VectorSubcoreMesh(core_axis_name="c", subcore_axis_name="s",
                                  num_cores=2, num_subcores=16)
    out_type = [jax.ShapeDtypeStruct((OUT_ROWS, DIM), jnp.float32)]
    scratch = [
        pltpu.VMEM((E_STAGE,), jnp.int32),
        pltpu.VMEM((E_STAGE,), jnp.int32),
        pltpu.VMEM((E_STAGE,), jnp.int32),
        pltpu.VMEM((COMP_ROWS, 128), jnp.int32),
        pltpu.VMEM((COMP_ROWS, 128), jnp.int32),
        pltpu.VMEM((128, DIM), jnp.float32),
        pltpu.VMEM((32, DIM), jnp.float32),
        pltpu.VMEM_SHARED((LOCAL_ROWS, DIM), jnp.float32),
    ]
    import functools as _ft
    return pl.kernel(_ft.partial(_sc_scatter_body, gather),
                     out_type=out_type, mesh=mesh, scratch_types=scratch,
                     compiler_params=pltpu.CompilerParams(
                         needs_layout_passes=False))


def _dense_body(x_ref, a_ref, cnt_ref, w_ref, b_ref, o_ref):
    x = x_ref[...]
    acc = jnp.dot(x, w_ref[0:DIM, :], preferred_element_type=jnp.float32)
    acc = acc + b_ref[0, :][None, :]
    for r in range(N_RELS):
        inv = 1.0 / jnp.maximum(cnt_ref[:, r], 1.0)
        ar = a_ref[:, r, :] * inv[:, None]
        acc = acc + jnp.dot(
            ar, w_ref[DIM * (r + 1):DIM * (r + 2), :],
            preferred_element_type=jnp.float32)
    o_ref[...] = jnp.maximum(acc, 0.0)


def _dense_layer(x, a, cnt, wcat, bias, *, interpret=False):
    return pl.pallas_call(
        _dense_body,
        grid=(N_SLOTS // _TC_BLK,),
        in_specs=[
            pl.BlockSpec((_TC_BLK, DIM), lambda i: (i, 0)),
            pl.BlockSpec((_TC_BLK, N_RELS, DIM), lambda i: (i, 0, 0)),
            pl.BlockSpec((_TC_BLK, N_RELS), lambda i: (i, 0)),
            pl.BlockSpec(((N_RELS + 1) * DIM, DIM), lambda i: (0, 0)),
            pl.BlockSpec((1, DIM), lambda i: (0, 0)),
        ],
        out_specs=pl.BlockSpec((_TC_BLK, DIM), lambda i: (i, 0)),
        out_shape=jax.ShapeDtypeStruct((N_SLOTS, DIM), jnp.float32),
        interpret=interpret,
    )(x, a, cnt, wcat, bias)


def kernel(x, edge_index, edge_type, W1, root1, b1, W2, root2, b2):
    src = edge_index[0].astype(jnp.int32)
    dst = edge_index[1].astype(jnp.int32)
    ety = edge_type.astype(jnp.int32)
    src_slot = (src // CHUNK_NODES) * CHUNK_SLOTS + src % CHUNK_NODES

    x_pad = jnp.pad(x.reshape(N_CHUNKS, CHUNK_NODES, DIM),
                    ((0, 0), (0, CHUNK_SLOTS - CHUNK_NODES), (0, 0))
                    ).reshape(N_SLOTS, DIM)

    w1 = jnp.concatenate([root1, W1.reshape(N_RELS * DIM, DIM)], axis=0)
    w2 = jnp.concatenate([root2, W2.reshape(N_RELS * DIM, DIM)], axis=0)
    bias1 = b1.reshape(1, DIM)
    bias2 = b2.reshape(1, DIM)

    z_h = jnp.zeros((32, DIM), jnp.float32)
    o_h = jnp.ones((128, DIM), jnp.float32)

    scat = _make_sc_scatter(True)
    scat_cnt = _make_sc_scatter(False)

    (cnt_wide,) = scat_cnt(x_pad, src_slot, dst, ety, z_h, o_h)
    cnt = cnt_wide[:, 0].reshape(N_SLOTS, N_RELS)

    (a1f,) = scat(x_pad, src_slot, dst, ety, z_h, o_h)
    a1 = a1f.reshape(N_SLOTS, N_RELS, DIM)
    h = _dense_layer(x_pad, a1, cnt, w1, bias1)

    (a2f,) = scat(h, src_slot, dst, ety, z_h, o_h)
    a2 = a2f.reshape(N_SLOTS, N_RELS, DIM)
    h2 = _dense_layer(h, a2, cnt, w2, bias2)

    return h2.reshape(N_CHUNKS, CHUNK_SLOTS, DIM)[:, :CHUNK_NODES].reshape(
        N_NODES, DIM)


# speedup vs baseline: 9.7700x; 1.1287x over previous
import jax
import jax.numpy as jnp
from jax import lax
from jax.experimental import pallas as pl
from jax.experimental.pallas import tpu as pltpu
from jax.experimental.pallas import tpu_sc as plsc

N_NODES = 10000
N_EDGES = 320000
DIM = 128
N_RELS = 8

N_CHUNKS = 10
CHUNK_NODES = 1000
CHUNK_SLOTS = 1024
N_SLOTS = N_CHUNKS * CHUNK_SLOTS
LOCAL_ROWS = CHUNK_SLOTS * N_RELS
OUT_ROWS = N_CHUNKS * LOCAL_ROWS
STRIPE = LOCAL_ROWS // 16
PASSES = N_CHUNKS // 2
E_PER_TILE = N_EDGES // 16
E_STAGE = 2000
N_STAGE = E_PER_TILE // E_STAGE
VECS_PER_STAGE = E_STAGE // 16
BATCH = 64
COMP_ROWS = 64
DUMMY_ROW = CHUNK_NODES * N_RELS + 100
DUMMY_SRC = CHUNK_NODES

_TC_BLK = 1280


def _sc_scatter_body(gather, x_pad, esrc, edst, ety, z_h, o_h, a_out,
                     est_s, est_d, est_t, csrc, ckey, rowbuf, zbuf, acc,
                     gsem, ssem):
    c = lax.axis_index("c")
    s = lax.axis_index("s")
    iota = lax.iota(jnp.int32, 16)

    pltpu.sync_copy(z_h, zbuf)

    rb0 = rowbuf.at[pl.ds(0, BATCH)]
    rb1 = rowbuf.at[pl.ds(BATCH, BATCH)]

    def drain(lo, hi):
        def gloop(k, carry):
            j0 = lo + 2 * k
            j1 = j0 + 1
            r0 = lax.bitwise_and(j0, COMP_ROWS - 1)
            r1 = lax.bitwise_and(j1, COMP_ROWS - 1)
            if gather:
                g0 = pltpu.async_copy(x_pad.at[csrc.at[r0]], rb0, gsem)

                @pl.when(j1 < hi)
                def _():
                    pltpu.async_copy(x_pad.at[csrc.at[r1]], rb1, gsem)
                g0.wait()
            s0 = pltpu.async_copy(rb0, acc.at[ckey.at[r0]], ssem, add=True)

            @pl.when(j1 < hi)
            def _():
                if gather:
                    pltpu.make_async_copy(
                        x_pad.at[csrc.at[r1]], rb1, gsem).wait()
                    s1 = pltpu.async_copy(
                        rb1, acc.at[ckey.at[r1]], ssem, add=True)
                else:
                    s1 = pltpu.async_copy(
                        rb0, acc.at[ckey.at[r1]], ssem, add=True)
                s1.wait()
            s0.wait()
            return carry
        npair = lax.shift_right_logical(hi - lo + 1, 1)
        lax.fori_loop(0, npair, gloop, 0)

    def pass_body(p, _):
        chunk = c * PASSES + p
        node_base = chunk * CHUNK_NODES
        out_base = chunk * LOCAL_ROWS
        if not gather:
            pltpu.sync_copy(o_h, rowbuf)

        def zloop(k, carry):
            pltpu.sync_copy(zbuf, acc.at[pl.ds(s * STRIPE + k * 32, 32)])
            return carry
        lax.fori_loop(0, STRIPE // 32, zloop, 0)
        plsc.subcore_barrier()

        def stage(b, carry):
            off, drained = carry
            e0 = s * E_PER_TILE + b * E_STAGE
            pltpu.sync_copy(esrc.at[pl.ds(e0, E_STAGE)], est_s)
            pltpu.sync_copy(edst.at[pl.ds(e0, E_STAGE)], est_d)
            pltpu.sync_copy(ety.at[pl.ds(e0, E_STAGE)], est_t)

            def iloop(i, off):
                for u in range(5):
                    sl = pl.ds((i * 5 + u) * 16, 16)
                    sv = est_s[sl]
                    dv = est_d[sl]
                    tv = est_t[sl]
                    dr = dv - node_base
                    m = (dr >= 0) & (dr < CHUNK_NODES)
                    key = dr * N_RELS + tv
                    pos = plsc.cumsum(jnp.where(m, 1, 0).astype(jnp.int32))
                    idx = off + pos - 1
                    row = lax.bitwise_and(
                        lax.shift_right_logical(idx, 6), COMP_ROWS - 1)
                    col = lax.bitwise_and(idx, BATCH - 1)
                    plsc.store_scatter(csrc, [row, col], sv, mask=m)
                    plsc.store_scatter(ckey, [row, col], key, mask=m)
                    off = off + plsc.all_reduce_population_count(m)
                return off
            off = lax.fori_loop(0, VECS_PER_STAGE // 5, iloop, off)
            full = lax.shift_right_logical(jnp.max(off), 6)
            drain(drained, full)
            return (off, full)
        off, drained = lax.fori_loop(
            0, N_STAGE, stage, (jnp.zeros((16,), jnp.int32),
                                jnp.zeros((), jnp.int32)))
        n_edges = jnp.max(off)
        nb = lax.shift_right_logical(n_edges + BATCH - 1, 6)

        def ploop(j, carry):
            posv = n_edges + j * 16 + iota
            pm = posv < nb * BATCH
            row = lax.bitwise_and(
                lax.shift_right_logical(posv, 6), COMP_ROWS - 1)
            col = lax.bitwise_and(posv, BATCH - 1)
            plsc.store_scatter(
                ckey, [row, col],
                jnp.full((16,), DUMMY_ROW, jnp.int32), mask=pm)
            plsc.store_scatter(
                csrc, [row, col],
                jnp.full((16,), DUMMY_SRC, jnp.int32), mask=pm)
            return carry
        lax.fori_loop(0, 4, ploop, 0)
        drain(drained, nb)
        plsc.subcore_barrier()

        def wloop(k, carry):
            r0 = s * STRIPE + k * 64
            rb = rowbuf.at[pl.ds(0, 64)]
            pltpu.sync_copy(acc.at[pl.ds(r0, 64)], rb)
            pltpu.sync_copy(rb, a_out.at[pl.ds(out_base + r0, 64)])
            return carry
        lax.fori_loop(0, STRIPE // 64, wloop, 0)
        return 0

    lax.fori_loop(0, PASSES, pass_body, 0)


def _make_sc_scatter(gather=True):
    mesh = plsc.VectorSubcoreMesh(core_axis_name="c", subcore_axis_name="s",
                                  num_cores=2, num_subcores=16)
    out_type = [jax.ShapeDtypeStruct((OUT_ROWS, DIM), jnp.float32)]
    scratch = [
        pltpu.VMEM((E_STAGE,), jnp.int32),
        pltpu.VMEM((E_STAGE,), jnp.int32),
        pltpu.VMEM((E_STAGE,), jnp.int32),
        pltpu.VMEM((COMP_ROWS, BATCH), jnp.int32),
        pltpu.VMEM((COMP_ROWS, BATCH), jnp.int32),
        pltpu.VMEM((128, DIM), jnp.float32),
        pltpu.VMEM((32, DIM), jnp.float32),
        pltpu.VMEM_SHARED((LOCAL_ROWS, DIM), jnp.float32),
        pltpu.SemaphoreType.DMA,
        pltpu.SemaphoreType.DMA,
    ]
    import functools as _ft
    return pl.kernel(_ft.partial(_sc_scatter_body, gather),
                     out_type=out_type, mesh=mesh, scratch_types=scratch,
                     compiler_params=pltpu.CompilerParams(
                         needs_layout_passes=False))


def _dense_body(x_ref, a_ref, cnt_ref, w_ref, b_ref, o_ref):
    x = x_ref[...]
    acc = jnp.dot(x, w_ref[0:DIM, :], preferred_element_type=jnp.float32)
    acc = acc + b_ref[0, :][None, :]
    for r in range(N_RELS):
        inv = 1.0 / jnp.maximum(cnt_ref[:, r], 1.0)
        ar = a_ref[:, r, :] * inv[:, None]
        acc = acc + jnp.dot(
            ar, w_ref[DIM * (r + 1):DIM * (r + 2), :],
            preferred_element_type=jnp.float32)
    o_ref[...] = jnp.maximum(acc, 0.0)


def _dense_layer(x, a, cnt, wcat, bias, *, interpret=False):
    return pl.pallas_call(
        _dense_body,
        grid=(N_SLOTS // _TC_BLK,),
        in_specs=[
            pl.BlockSpec((_TC_BLK, DIM), lambda i: (i, 0)),
            pl.BlockSpec((_TC_BLK, N_RELS, DIM), lambda i: (i, 0, 0)),
            pl.BlockSpec((_TC_BLK, N_RELS), lambda i: (i, 0)),
            pl.BlockSpec(((N_RELS + 1) * DIM, DIM), lambda i: (0, 0)),
            pl.BlockSpec((1, DIM), lambda i: (0, 0)),
        ],
        out_specs=pl.BlockSpec((_TC_BLK, DIM), lambda i: (i, 0)),
        out_shape=jax.ShapeDtypeStruct((N_SLOTS, DIM), jnp.float32),
        interpret=interpret,
    )(x, a, cnt, wcat, bias)


def kernel(x, edge_index, edge_type, W1, root1, b1, W2, root2, b2):
    src = edge_index[0].astype(jnp.int32)
    dst = edge_index[1].astype(jnp.int32)
    ety = edge_type.astype(jnp.int32)
    src_slot = (src // CHUNK_NODES) * CHUNK_SLOTS + src % CHUNK_NODES

    x_pad = jnp.pad(x.reshape(N_CHUNKS, CHUNK_NODES, DIM),
                    ((0, 0), (0, CHUNK_SLOTS - CHUNK_NODES), (0, 0))
                    ).reshape(N_SLOTS, DIM)

    w1 = jnp.concatenate([root1, W1.reshape(N_RELS * DIM, DIM)], axis=0)
    w2 = jnp.concatenate([root2, W2.reshape(N_RELS * DIM, DIM)], axis=0)
    bias1 = b1.reshape(1, DIM)
    bias2 = b2.reshape(1, DIM)

    z_h = jnp.zeros((32, DIM), jnp.float32)
    o_h = jnp.ones((128, DIM), jnp.float32)

    scat = _make_sc_scatter(True)
    scat_cnt = _make_sc_scatter(False)

    (cnt_wide,) = scat_cnt(x_pad, src_slot, dst, ety, z_h, o_h)
    cnt = cnt_wide[:, 0].reshape(N_SLOTS, N_RELS)

    (a1f,) = scat(x_pad, src_slot, dst, ety, z_h, o_h)
    a1 = a1f.reshape(N_SLOTS, N_RELS, DIM)
    h = _dense_layer(x_pad, a1, cnt, w1, bias1)

    (a2f,) = scat(h, src_slot, dst, ety, z_h, o_h)
    a2 = a2f.reshape(N_SLOTS, N_RELS, DIM)
    h2 = _dense_layer(h, a2, cnt, w2, bias2)

    return h2.reshape(N_CHUNKS, CHUNK_SLOTS, DIM)[:, :CHUNK_NODES].reshape(
        N_NODES, DIM)


# speedup vs baseline: 9.8616x; 1.0094x over previous
import jax
import jax.numpy as jnp
from jax import lax
from jax.experimental import pallas as pl
from jax.experimental.pallas import tpu as pltpu
from jax.experimental.pallas import tpu_sc as plsc

N_NODES = 10000
N_EDGES = 320000
DIM = 128
N_RELS = 8

N_CHUNKS = 10
CHUNK_NODES = 1000
CHUNK_SLOTS = 1024
N_SLOTS = N_CHUNKS * CHUNK_SLOTS
LOCAL_ROWS = CHUNK_SLOTS * N_RELS
OUT_ROWS = N_CHUNKS * LOCAL_ROWS
STRIPE = LOCAL_ROWS // 16
PASSES = N_CHUNKS // 2
E_PER_TILE = N_EDGES // 16
E_STAGE = 2000
N_STAGE = E_PER_TILE // E_STAGE
VECS_PER_STAGE = E_STAGE // 16
BATCH = 64
COMP_ROWS = 64
DUMMY_ROW = CHUNK_NODES * N_RELS + 100
DUMMY_SRC = CHUNK_NODES

_TC_BLK = 1280


def _sc_scatter_body(gather, x_pad, esrc, edst, ety, z_h, o_h, a_out,
                     est_s, est_d, est_t, csrc, ckey, rowbuf, zbuf, acc,
                     gsem, ssem):
    c = lax.axis_index("c")
    s = lax.axis_index("s")
    iota = lax.iota(jnp.int32, 16)

    pltpu.sync_copy(z_h, zbuf)

    rb0 = rowbuf.at[pl.ds(0, BATCH)]
    rb1 = rowbuf.at[pl.ds(BATCH, BATCH)]

    def drain(lo, hi):
        def gloop(k, carry):
            j0 = lo + 2 * k
            j1 = j0 + 1
            r0 = lax.bitwise_and(j0, COMP_ROWS - 1)
            r1 = lax.bitwise_and(j1, COMP_ROWS - 1)
            if gather:
                g0 = pltpu.async_copy(x_pad.at[csrc.at[r0]], rb0, gsem)

                @pl.when(j1 < hi)
                def _():
                    pltpu.async_copy(x_pad.at[csrc.at[r1]], rb1, gsem)
                g0.wait()
            s0 = pltpu.async_copy(rb0, acc.at[ckey.at[r0]], ssem, add=True)

            @pl.when(j1 < hi)
            def _():
                if gather:
                    pltpu.make_async_copy(
                        x_pad.at[csrc.at[r1]], rb1, gsem).wait()
                    s1 = pltpu.async_copy(
                        rb1, acc.at[ckey.at[r1]], ssem, add=True)
                else:
                    s1 = pltpu.async_copy(
                        rb0, acc.at[ckey.at[r1]], ssem, add=True)
                s1.wait()
            s0.wait()
            return carry
        npair = lax.shift_right_logical(hi - lo + 1, 1)
        lax.fori_loop(0, npair, gloop, 0)

    def pass_body(p, _):
        chunk = c * PASSES + p
        node_base = chunk * CHUNK_NODES
        out_base = chunk * LOCAL_ROWS
        if not gather:
            pltpu.sync_copy(o_h, rowbuf)

        def zloop(k, carry):
            pltpu.sync_copy(zbuf, acc.at[pl.ds(s * STRIPE + k * 32, 32)])
            return carry
        lax.fori_loop(0, STRIPE // 32, zloop, 0)
        plsc.subcore_barrier()

        def stage(b, carry):
            off, drained = carry
            e0 = s * E_PER_TILE + b * E_STAGE
            pltpu.sync_copy(esrc.at[pl.ds(e0, E_STAGE)], est_s)
            pltpu.sync_copy(edst.at[pl.ds(e0, E_STAGE)], est_d)
            pltpu.sync_copy(ety.at[pl.ds(e0, E_STAGE)], est_t)

            def iloop(i, off):
                for u in range(5):
                    sl = pl.ds((i * 5 + u) * 16, 16)
                    sv = est_s[sl]
                    dv = est_d[sl]
                    tv = est_t[sl]
                    dr = dv - node_base
                    m = (dr >= 0) & (dr < CHUNK_NODES)
                    key = dr * N_RELS + tv
                    pos = plsc.cumsum(jnp.where(m, 1, 0).astype(jnp.int32))
                    idx = off + pos - 1
                    row = lax.bitwise_and(
                        lax.shift_right_logical(idx, 6), COMP_ROWS - 1)
                    col = lax.bitwise_and(idx, BATCH - 1)
                    plsc.store_scatter(csrc, [row, col], sv, mask=m)
                    plsc.store_scatter(ckey, [row, col], key, mask=m)
                    off = off + plsc.all_reduce_population_count(m)
                return off
            off = lax.fori_loop(0, VECS_PER_STAGE // 5, iloop, off)
            full = lax.shift_right_logical(jnp.max(off), 6)
            drain(drained, full)
            return (off, full)
        off, drained = lax.fori_loop(
            0, N_STAGE, stage, (jnp.zeros((16,), jnp.int32),
                                jnp.zeros((), jnp.int32)))
        n_edges = jnp.max(off)
        nb = lax.shift_right_logical(n_edges + BATCH - 1, 6)

        def ploop(j, carry):
            posv = n_edges + j * 16 + iota
            pm = posv < nb * BATCH
            row = lax.bitwise_and(
                lax.shift_right_logical(posv, 6), COMP_ROWS - 1)
            col = lax.bitwise_and(posv, BATCH - 1)
            plsc.store_scatter(
                ckey, [row, col],
                jnp.full((16,), DUMMY_ROW, jnp.int32), mask=pm)
            plsc.store_scatter(
                csrc, [row, col],
                jnp.full((16,), DUMMY_SRC, jnp.int32), mask=pm)
            return carry
        lax.fori_loop(0, 4, ploop, 0)
        drain(drained, nb)
        plsc.subcore_barrier()

        def wloop(k, carry):
            r0 = s * STRIPE + k * 128
            w0 = pltpu.async_copy(acc.at[pl.ds(r0, 64)],
                                  a_out.at[pl.ds(out_base + r0, 64)], gsem)
            w1 = pltpu.async_copy(acc.at[pl.ds(r0 + 64, 64)],
                                  a_out.at[pl.ds(out_base + r0 + 64, 64)],
                                  gsem)
            w0.wait()
            w1.wait()
            return carry
        lax.fori_loop(0, STRIPE // 128, wloop, 0)
        return 0

    lax.fori_loop(0, PASSES, pass_body, 0)


def _make_sc_scatter(gather=True):
    mesh = plsc.VectorSubcoreMesh(core_axis_name="c", subcore_axis_name="s",
                                  num_cores=2, num_subcores=16)
    out_type = [jax.ShapeDtypeStruct((OUT_ROWS, DIM), jnp.float32)]
    scratch = [
        pltpu.VMEM((E_STAGE,), jnp.int32),
        pltpu.VMEM((E_STAGE,), jnp.int32),
        pltpu.VMEM((E_STAGE,), jnp.int32),
        pltpu.VMEM((COMP_ROWS, BATCH), jnp.int32),
        pltpu.VMEM((COMP_ROWS, BATCH), jnp.int32),
        pltpu.VMEM((128, DIM), jnp.float32),
        pltpu.VMEM((32, DIM), jnp.float32),
        pltpu.VMEM_SHARED((LOCAL_ROWS, DIM), jnp.float32),
        pltpu.SemaphoreType.DMA,
        pltpu.SemaphoreType.DMA,
    ]
    import functools as _ft
    return pl.kernel(_ft.partial(_sc_scatter_body, gather),
                     out_type=out_type, mesh=mesh, scratch_types=scratch,
                     compiler_params=pltpu.CompilerParams(
                         needs_layout_passes=False))


def _dense_body(x_ref, a_ref, cnt_ref, w_ref, b_ref, o_ref):
    x = x_ref[...]
    acc = jnp.dot(x, w_ref[0:DIM, :], preferred_element_type=jnp.float32)
    acc = acc + b_ref[0, :][None, :]
    for r in range(N_RELS):
        inv = 1.0 / jnp.maximum(cnt_ref[:, r], 1.0)
        ar = a_ref[:, r, :] * inv[:, None]
        acc = acc + jnp.dot(
            ar, w_ref[DIM * (r + 1):DIM * (r + 2), :],
            preferred_element_type=jnp.float32)
    o_ref[...] = jnp.maximum(acc, 0.0)


def _dense_layer(x, a, cnt, wcat, bias, *, interpret=False):
    return pl.pallas_call(
        _dense_body,
        grid=(N_SLOTS // _TC_BLK,),
        in_specs=[
            pl.BlockSpec((_TC_BLK, DIM), lambda i: (i, 0)),
            pl.BlockSpec((_TC_BLK, N_RELS, DIM), lambda i: (i, 0, 0)),
            pl.BlockSpec((_TC_BLK, N_RELS), lambda i: (i, 0)),
            pl.BlockSpec(((N_RELS + 1) * DIM, DIM), lambda i: (0, 0)),
            pl.BlockSpec((1, DIM), lambda i: (0, 0)),
        ],
        out_specs=pl.BlockSpec((_TC_BLK, DIM), lambda i: (i, 0)),
        out_shape=jax.ShapeDtypeStruct((N_SLOTS, DIM), jnp.float32),
        interpret=interpret,
    )(x, a, cnt, wcat, bias)


def kernel(x, edge_index, edge_type, W1, root1, b1, W2, root2, b2):
    src = edge_index[0].astype(jnp.int32)
    dst = edge_index[1].astype(jnp.int32)
    ety = edge_type.astype(jnp.int32)
    src_slot = (src // CHUNK_NODES) * CHUNK_SLOTS + src % CHUNK_NODES

    x_pad = jnp.pad(x.reshape(N_CHUNKS, CHUNK_NODES, DIM),
                    ((0, 0), (0, CHUNK_SLOTS - CHUNK_NODES), (0, 0))
                    ).reshape(N_SLOTS, DIM)

    w1 = jnp.concatenate([root1, W1.reshape(N_RELS * DIM, DIM)], axis=0)
    w2 = jnp.concatenate([root2, W2.reshape(N_RELS * DIM, DIM)], axis=0)
    bias1 = b1.reshape(1, DIM)
    bias2 = b2.reshape(1, DIM)

    z_h = jnp.zeros((32, DIM), jnp.float32)
    o_h = jnp.ones((128, DIM), jnp.float32)

    scat = _make_sc_scatter(True)
    scat_cnt = _make_sc_scatter(False)

    (cnt_wide,) = scat_cnt(x_pad, src_slot, dst, ety, z_h, o_h)
    cnt = cnt_wide[:, 0].reshape(N_SLOTS, N_RELS)

    (a1f,) = scat(x_pad, src_slot, dst, ety, z_h, o_h)
    a1 = a1f.reshape(N_SLOTS, N_RELS, DIM)
    h = _dense_layer(x_pad, a1, cnt, w1, bias1)

    (a2f,) = scat(h, src_slot, dst, ety, z_h, o_h)
    a2 = a2f.reshape(N_SLOTS, N_RELS, DIM)
    h2 = _dense_layer(h, a2, cnt, w2, bias2)

    return h2.reshape(N_CHUNKS, CHUNK_SLOTS, DIM)[:, :CHUNK_NODES].reshape(
        N_NODES, DIM)
